# baseline (speedup 1.0000x reference)
import functools

import jax
import jax.numpy as jnp
from jax import lax
from jax.experimental import pallas as pl
from jax.experimental.pallas import tpu as pltpu
from jax.experimental.pallas import tpu_sc as plsc

VOCAB = 100000
HIDDEN = 128
N_TOK = 512

_NC = 2
_NS = 16
_NW = _NC * _NS
_TOK_PER_W = N_TOK // _NW

_mesh = plsc.VectorSubcoreMesh(core_axis_name="c", subcore_axis_name="s")


@functools.partial(
    pl.kernel,
    out_type=jax.ShapeDtypeStruct((N_TOK, HIDDEN), jnp.float32),
    mesh=_mesh,
    scratch_types=[
        pltpu.VMEM((_TOK_PER_W,), jnp.int32),
        pltpu.VMEM((_TOK_PER_W, HIDDEN), jnp.float32),
        pltpu.SemaphoreType.DMA,
    ],
)
def _sc_gather(table_hbm, idx_hbm, out_hbm, idx_v, rows_v, sem):
    wid = lax.axis_index("s") * _NC + lax.axis_index("c")
    base = wid * _TOK_PER_W
    pltpu.sync_copy(idx_hbm.at[pl.ds(base, _TOK_PER_W)], idx_v)
    pltpu.async_copy(table_hbm.at[idx_v], rows_v, sem).wait()
    pltpu.sync_copy(rows_v, out_hbm.at[pl.ds(base, _TOK_PER_W)])


_VT = 4096


def _proj_body(h_ref, w_ref, b_ref, o_ref):
    h = h_ref[...].astype(jnp.bfloat16)
    w = w_ref[...].astype(jnp.bfloat16)
    acc = lax.dot_general(
        h, w, (((1,), (1,)), ((), ())), preferred_element_type=jnp.float32
    )
    o_ref[...] = acc + b_ref[...]


def _projection(h, W, b2):
    grid = (pl.cdiv(VOCAB, _VT),)
    return pl.pallas_call(
        _proj_body,
        grid=grid,
        in_specs=[
            pl.BlockSpec((N_TOK, HIDDEN), lambda v: (0, 0)),
            pl.BlockSpec((_VT, HIDDEN), lambda v: (v, 0)),
            pl.BlockSpec((1, _VT), lambda v: (0, v)),
        ],
        out_specs=pl.BlockSpec((N_TOK, _VT), lambda v: (0, v)),
        out_shape=jax.ShapeDtypeStruct((N_TOK, VOCAB), jnp.float32),
        compiler_params=pltpu.CompilerParams(
            dimension_semantics=("arbitrary",),
        ),
    )(h, W, b2)


def kernel(x, we, W, b):
    bsz, seq = x.shape
    idx = x.reshape(N_TOK).astype(jnp.int32)
    h = _sc_gather(we, idx)
    out = _projection(h, W, b.reshape(1, VOCAB))
    return out.reshape(bsz, seq, VOCAB)


# speedup vs baseline: 1.0152x; 1.0152x over previous
import functools

import jax
import jax.numpy as jnp
from jax import lax
from jax.experimental import pallas as pl
from jax.experimental.pallas import tpu as pltpu
from jax.experimental.pallas import tpu_sc as plsc

VOCAB = 100000
HIDDEN = 128
N_TOK = 512

_NC = 2
_NS = 16
_NW = _NC * _NS
_TOK_PER_W = N_TOK // _NW

_mesh = plsc.VectorSubcoreMesh(core_axis_name="c", subcore_axis_name="s")


@functools.partial(
    pl.kernel,
    out_type=jax.ShapeDtypeStruct((N_TOK, HIDDEN), jnp.float32),
    mesh=_mesh,
    scratch_types=[
        pltpu.VMEM((_TOK_PER_W,), jnp.int32),
        pltpu.VMEM((_TOK_PER_W, HIDDEN), jnp.float32),
        pltpu.SemaphoreType.DMA,
    ],
)
def _sc_gather(table_hbm, idx_hbm, out_hbm, idx_v, rows_v, sem):
    wid = lax.axis_index("s") * _NC + lax.axis_index("c")
    base = wid * _TOK_PER_W
    pltpu.sync_copy(idx_hbm.at[pl.ds(base, _TOK_PER_W)], idx_v)
    pltpu.async_copy(table_hbm.at[idx_v], rows_v, sem).wait()
    pltpu.sync_copy(rows_v, out_hbm.at[pl.ds(base, _TOK_PER_W)])


_VT = 6272


def _proj_body(h_ref, w_ref, b_ref, o_ref):
    h = h_ref[...].astype(jnp.bfloat16)
    w = w_ref[...].astype(jnp.bfloat16)
    acc = lax.dot_general(
        h, w, (((1,), (1,)), ((), ())), preferred_element_type=jnp.float32
    )
    o_ref[...] = acc + b_ref[...]


def _projection(h, W, b2):
    grid = (pl.cdiv(VOCAB, _VT),)
    return pl.pallas_call(
        _proj_body,
        grid=grid,
        in_specs=[
            pl.BlockSpec((N_TOK, HIDDEN), lambda v: (0, 0)),
            pl.BlockSpec((_VT, HIDDEN), lambda v: (v, 0)),
            pl.BlockSpec((1, _VT), lambda v: (0, v)),
        ],
        out_specs=pl.BlockSpec((N_TOK, _VT), lambda v: (0, v)),
        out_shape=jax.ShapeDtypeStruct((N_TOK, VOCAB), jnp.float32),
        compiler_params=pltpu.CompilerParams(
            dimension_semantics=("arbitrary",),
        ),
    )(h, W, b2)


def kernel(x, we, W, b):
    bsz, seq = x.shape
    idx = x.reshape(N_TOK).astype(jnp.int32)
    h = _sc_gather(we, idx)
    out = _projection(h, W, b.reshape(1, VOCAB))
    return out.reshape(bsz, seq, VOCAB)


# speedup vs baseline: 1.0176x; 1.0024x over previous
import functools

import jax
import jax.numpy as jnp
from jax import lax
from jax.experimental import pallas as pl
from jax.experimental.pallas import tpu as pltpu
from jax.experimental.pallas import tpu_sc as plsc

VOCAB = 100000
HIDDEN = 128
N_TOK = 512

_NC = 2
_NS = 16
_NW = _NC * _NS
_TOK_PER_W = N_TOK // _NW

_mesh = plsc.VectorSubcoreMesh(core_axis_name="c", subcore_axis_name="s")


@functools.partial(
    pl.kernel,
    out_type=jax.ShapeDtypeStruct((N_TOK, HIDDEN), jnp.float32),
    mesh=_mesh,
    scratch_types=[
        pltpu.VMEM((_TOK_PER_W,), jnp.int32),
        pltpu.VMEM((_TOK_PER_W, HIDDEN), jnp.float32),
        pltpu.SemaphoreType.DMA,
    ],
)
def _sc_gather(table_hbm, idx_hbm, out_hbm, idx_v, rows_v, sem):
    wid = lax.axis_index("s") * _NC + lax.axis_index("c")
    base = wid * _TOK_PER_W
    pltpu.sync_copy(idx_hbm.at[pl.ds(base, _TOK_PER_W)], idx_v)
    pltpu.async_copy(table_hbm.at[idx_v], rows_v, sem).wait()
    pltpu.sync_copy(rows_v, out_hbm.at[pl.ds(base, _TOK_PER_W)])


_VT = 8192


def _proj_body(h_ref, w_ref, b_ref, o_ref):
    h = h_ref[...].astype(jnp.bfloat16)
    w = w_ref[...].astype(jnp.bfloat16)
    acc = lax.dot_general(
        h, w, (((1,), (1,)), ((), ())), preferred_element_type=jnp.float32
    )
    o_ref[...] = acc + b_ref[...]


def _projection(h, W, b2):
    grid = (pl.cdiv(VOCAB, _VT),)
    return pl.pallas_call(
        _proj_body,
        grid=grid,
        in_specs=[
            pl.BlockSpec((N_TOK, HIDDEN), lambda v: (0, 0)),
            pl.BlockSpec((_VT, HIDDEN), lambda v: (v, 0)),
            pl.BlockSpec((1, _VT), lambda v: (0, v)),
        ],
        out_specs=pl.BlockSpec((N_TOK, _VT), lambda v: (0, v)),
        out_shape=jax.ShapeDtypeStruct((N_TOK, VOCAB), jnp.float32),
        compiler_params=pltpu.CompilerParams(
            dimension_semantics=("arbitrary",),
        ),
    )(h, W, b2)


def kernel(x, we, W, b):
    bsz, seq = x.shape
    idx = x.reshape(N_TOK).astype(jnp.int32)
    h = _sc_gather(we, idx)
    out = _projection(h, W, b.reshape(1, VOCAB))
    return out.reshape(bsz, seq, VOCAB)


# speedup vs baseline: 1.1727x; 1.1524x over previous
import jax
import jax.numpy as jnp
from jax import lax
from jax.experimental import pallas as pl
from jax.experimental.pallas import tpu as pltpu

VOCAB = 100000
HIDDEN = 128
N_TOK = 512

_VT = 8192


def _body(idx_ref, we_ref, w_ref, b_ref, o_ref, h_raw, h_bf, sem):
    v = pl.program_id(0)

    @pl.when(v == 0)
    def _gather():
        def issue(i, _):
            pltpu.make_async_copy(
                we_ref.at[pl.ds(idx_ref[i], 1), :], h_raw.at[pl.ds(i, 1), :], sem
            ).start()
            return 0

        lax.fori_loop(0, N_TOK, issue, 0, unroll=8)

        def drain(i, _):
            pltpu.make_async_copy(
                we_ref.at[pl.ds(0, 1), :], h_raw.at[pl.ds(0, 1), :], sem
            ).wait()
            return 0

        lax.fori_loop(0, N_TOK, drain, 0, unroll=8)
        h_bf[...] = h_raw[...].astype(jnp.bfloat16)

    w = w_ref[...].astype(jnp.bfloat16)
    acc = lax.dot_general(
        h_bf[...], w, (((1,), (1,)), ((), ())), preferred_element_type=jnp.float32
    )
    o_ref[...] = acc + b_ref[...]


def kernel(x, we, W, b):
    bsz, seq = x.shape
    idx = x.reshape(N_TOK).astype(jnp.int32)
    grid = (pl.cdiv(VOCAB, _VT),)
    out = pl.pallas_call(
        _body,
        grid_spec=pltpu.PrefetchScalarGridSpec(
            num_scalar_prefetch=1,
            grid=grid,
            in_specs=[
                pl.BlockSpec(memory_space=pltpu.HBM),
                pl.BlockSpec((_VT, HIDDEN), lambda v, idx: (v, 0)),
                pl.BlockSpec((1, _VT), lambda v, idx: (0, v)),
            ],
            out_specs=pl.BlockSpec((N_TOK, _VT), lambda v, idx: (0, v)),
            scratch_shapes=[
                pltpu.VMEM((N_TOK, HIDDEN), jnp.float32),
                pltpu.VMEM((N_TOK, HIDDEN), jnp.bfloat16),
                pltpu.SemaphoreType.DMA,
            ],
        ),
        out_shape=jax.ShapeDtypeStruct((N_TOK, VOCAB), jnp.float32),
        compiler_params=pltpu.CompilerParams(
            dimension_semantics=("arbitrary",),
        ),
    )(idx, we, W, b.reshape(1, VOCAB))
    return out.reshape(bsz, seq, VOCAB)
